# baseline (speedup 1.0000x reference)
import jax
import jax.numpy as jnp
import numpy as np
from jax import lax
from jax.experimental import pallas as pl
from jax.experimental.pallas import tpu as pltpu
from jax.experimental.pallas import tpu_sc as plsc

VOCAB = 1000000
DIM = 64
B = 4096
L = 200
N_SEG = 3

NC = 2
NS = 16
NW = NC * NS
LANES = 16

ROWS_PW = B // NW
NBUF = 2


def _sinusoidal_pe(length, dim):
    pos = np.arange(length)[:, None].astype(np.float64)
    i = np.arange(dim)[None, :]
    angle_rates = 1.0 / np.power(10000.0, (2 * (i // 2)) / np.float64(dim))
    angles = pos * angle_rates
    pe = np.zeros((length, dim), dtype=np.float64)
    pe[:, 0::2] = np.sin(angles[:, 0::2])
    pe[:, 1::2] = np.cos(angles[:, 1::2])
    return pe.astype(np.float32)


_PE_T = _sinusoidal_pe(L, DIM).T.copy()


def _combine_kernel(pe_ref, seg_ref, ct_ref):
    ct_ref[...] = pe_ref[...][None, :, :] + seg_ref[...][:, :, None]


@jax.jit
def _combine(pe_t, seg_table):
    ct = pl.pallas_call(
        _combine_kernel,
        out_shape=jax.ShapeDtypeStruct((N_SEG, DIM, L), jnp.float32),
    )(pe_t, seg_table)
    return ct.reshape(N_SEG * DIM * L)


GROUP_BASES = [16 * g for g in range(12)] + [L - LANES]
STRIDE = 65


def _dup_kernel(tok_ref, out_ref):
    t = tok_ref[...]
    out_ref[...] = jnp.concatenate([t, t], axis=1)


@jax.jit
def _dup(token_table):
    blk = 25000
    return pl.pallas_call(
        _dup_kernel,
        grid=(VOCAB // blk,),
        in_specs=[pl.BlockSpec((blk, DIM), lambda i: (i, 0))],
        out_specs=pl.BlockSpec((blk, 2 * DIM), lambda i: (i, 0)),
        out_shape=jax.ShapeDtypeStruct((VOCAB, 2 * DIM), jnp.float32),
    )(token_table)


def _sc_body(seq_hbm, lbl_hbm, tok_hbm, ct_hbm, out_hbm, *scr):
    idx_v = scr[0:NBUF]
    lbl_v = scr[NBUF:2 * NBUF]
    rows_v = scr[2 * NBUF:3 * NBUF]
    pan_v = scr[3 * NBUF]
    ct_v = scr[3 * NBUF + 1]
    tok65_v = scr[3 * NBUF + 2]
    fsem = scr[3 * NBUF + 3:3 * NBUF + 3 + NBUF]
    gsem = scr[3 * NBUF + 3 + NBUF:3 * NBUF + 3 + 2 * NBUF]
    ssem = scr[3 * NBUF + 3 + 2 * NBUF]

    wid = lax.axis_index("s") * NC + lax.axis_index("c")
    rbase = wid * ROWS_PW

    pltpu.sync_copy(ct_hbm, ct_v)

    @pl.loop(0, ROWS_PW, step=NBUF)
    def iteration(k0):
        rows = [rbase + k0 + b for b in range(NBUF)]

        for b in range(NBUF):
            off = rows[b] * L
            pltpu.async_copy(seq_hbm.at[pl.ds(off, L)], idx_v[b], fsem[b])
            pltpu.async_copy(lbl_hbm.at[pl.ds(off, L)], lbl_v[b], fsem[b])

        for b in range(NBUF):
            off = rows[b] * L
            pltpu.make_async_copy(
                seq_hbm.at[pl.ds(off, L)], idx_v[b], fsem[b]).wait()
            pltpu.make_async_copy(
                lbl_hbm.at[pl.ds(off, L)], lbl_v[b], fsem[b]).wait()
            pltpu.async_copy(tok_hbm.at[idx_v[b].at[pl.ds(0, 128)]],
                             rows_v[b].at[pl.ds(0, 128)], gsem[b])
            pltpu.async_copy(tok_hbm.at[idx_v[b].at[pl.ds(128, L - 128)]],
                             rows_v[b].at[pl.ds(128, L - 128)], gsem[b])

        for b in range(NBUF):
            pltpu.make_async_copy(
                tok_hbm.at[idx_v[b].at[pl.ds(0, 128)]],
                rows_v[b].at[pl.ds(0, 128)], gsem[b]).wait()
            pltpu.make_async_copy(
                tok_hbm.at[idx_v[b].at[pl.ds(128, L - 128)]],
                rows_v[b].at[pl.ds(128, L - 128)], gsem[b]).wait()

            @plsc.parallel_loop(0, L, unroll=2)
            def restage(r, b=b):
                rb = r * STRIDE
                for d in range(DIM // LANES):
                    tok65_v[pl.ds(rb + d * LANES, LANES)] = (
                        rows_v[b][r, pl.ds(d * LANES, LANES)])

            if b == 0:
                @pl.when(k0 > 0)
                def _drain():
                    pltpu.make_async_copy(
                        pan_v, out_hbm.at[rows[0]], ssem).wait()
            else:
                pltpu.make_async_copy(
                    pan_v, out_hbm.at[rows[b]], ssem).wait()

            @pl.loop(0, len(GROUP_BASES))
            def per_group(g, b=b):
                gb = lax.min(g * LANES, jnp.int32(L - LANES))
                sl = pl.ds(gb, LANES)
                lvec = gb + lax.iota(jnp.int32, LANES)
                tbase = lvec * STRIDE
                cbase = lbl_v[b][sl] * (DIM * L) + lvec
                for dd in range(DIM):
                    t16 = plsc.load_gather(tok65_v, [tbase + dd])
                    c16 = plsc.load_gather(ct_v, [cbase + dd * L])
                    pan_v[dd, sl] = t16 + c16

            pltpu.async_copy(pan_v, out_hbm.at[rows[b]], ssem)

    pltpu.make_async_copy(
        pan_v, out_hbm.at[rbase + ROWS_PW - 1], ssem).wait()


@jax.jit
def _sc_embed(seq_flat, lbl_flat, tok128, ct_flat):
    mesh = plsc.VectorSubcoreMesh(core_axis_name="c", subcore_axis_name="s")
    return pl.kernel(
        _sc_body,
        out_type=jax.ShapeDtypeStruct((B, DIM, L), jnp.float32),
        mesh=mesh,
        compiler_params=pltpu.CompilerParams(
            use_tc_tiling_on_sc=True, needs_layout_passes=False,
            disable_bounds_checks=True),
        scratch_types=(
            [pltpu.VMEM((L,), jnp.int32)] * NBUF
            + [pltpu.VMEM((L,), jnp.int32)] * NBUF
            + [pltpu.VMEM((L, 128), jnp.float32)] * NBUF
            + [pltpu.VMEM((DIM, L), jnp.float32)]
            + [pltpu.VMEM((N_SEG * DIM * L,), jnp.float32)]
            + [pltpu.VMEM((L * STRIDE,), jnp.float32)]
            + [pltpu.SemaphoreType.DMA] * (2 * NBUF + 1)
        ),
    )(seq_flat, lbl_flat, tok128, ct_flat)


def kernel(seq, seg_label, token_table, seg_table):
    pe_t = jnp.asarray(_PE_T)
    ct_flat = _combine(pe_t, seg_table)
    tok128 = _dup(token_table)
    out_dl = _sc_embed(
        seq.reshape(B * L).astype(jnp.int32),
        seg_label.reshape(B * L).astype(jnp.int32),
        tok128,
        ct_flat,
    )
    return out_dl.transpose(0, 2, 1)


# speedup vs baseline: 2.4734x; 2.4734x over previous
import jax
import jax.numpy as jnp
import numpy as np
from jax import lax
from jax.experimental import pallas as pl
from jax.experimental.pallas import tpu as pltpu
from jax.experimental.pallas import tpu_sc as plsc

VOCAB = 1000000
DIM = 64
B = 4096
L = 200
N_SEG = 3

NC = 2
NS = 16
NW = NC * NS
LANES = 16

ROWS_PW = B // NW
NBUF = 2


def _sinusoidal_pe(length, dim):
    pos = np.arange(length)[:, None].astype(np.float64)
    i = np.arange(dim)[None, :]
    angle_rates = 1.0 / np.power(10000.0, (2 * (i // 2)) / np.float64(dim))
    angles = pos * angle_rates
    pe = np.zeros((length, dim), dtype=np.float64)
    pe[:, 0::2] = np.sin(angles[:, 0::2])
    pe[:, 1::2] = np.cos(angles[:, 1::2])
    return pe.astype(np.float32)


_PE_T = _sinusoidal_pe(L, DIM).T.copy()


def _combine_kernel(pe_ref, seg_ref, ct_ref):
    ct_ref[...] = pe_ref[...][None, :, :] + seg_ref[...][:, :, None]


@jax.jit
def _combine(pe_t, seg_table):
    ct = pl.pallas_call(
        _combine_kernel,
        out_shape=jax.ShapeDtypeStruct((N_SEG, DIM, L), jnp.float32),
    )(pe_t, seg_table)
    return ct.reshape(N_SEG * DIM * L)


GROUP_BASES = [16 * g for g in range(12)] + [L - LANES]
STRIDE = 65


def _dup_kernel(tok_ref, out_ref):
    t = tok_ref[...]
    out_ref[...] = jnp.concatenate([t, t], axis=1)


@jax.jit
def _dup(token_table):
    blk = 25000
    return pl.pallas_call(
        _dup_kernel,
        grid=(VOCAB // blk,),
        in_specs=[pl.BlockSpec((blk, DIM), lambda i: (i, 0))],
        out_specs=pl.BlockSpec((blk, 2 * DIM), lambda i: (i, 0)),
        out_shape=jax.ShapeDtypeStruct((VOCAB, 2 * DIM), jnp.float32),
    )(token_table)


def _sc_body(seq_hbm, lbl_hbm, tok_hbm, ct_hbm, out_hbm, *scr):
    idx_v = scr[0:NBUF]
    lbl_v = scr[NBUF:2 * NBUF]
    rows_v = scr[2 * NBUF:3 * NBUF]
    pan_v = scr[3 * NBUF]
    ct_v = scr[3 * NBUF + 1]
    tok65_v = scr[3 * NBUF + 2]
    fsem = scr[3 * NBUF + 3:3 * NBUF + 3 + NBUF]
    gsem = scr[3 * NBUF + 3 + NBUF:3 * NBUF + 3 + 2 * NBUF]
    ssem = scr[3 * NBUF + 3 + 2 * NBUF]

    wid = lax.axis_index("s") * NC + lax.axis_index("c")
    rbase = wid * ROWS_PW

    pltpu.sync_copy(ct_hbm, ct_v)

    @pl.loop(0, ROWS_PW, step=NBUF)
    def iteration(k0):
        rows = [rbase + k0 + b for b in range(NBUF)]

        for b in range(NBUF):
            off = rows[b] * L
            pltpu.async_copy(seq_hbm.at[pl.ds(off, L)], idx_v[b], fsem[b])
            pltpu.async_copy(lbl_hbm.at[pl.ds(off, L)], lbl_v[b], fsem[b])

        for b in range(NBUF):
            off = rows[b] * L
            with jax.named_scope("fwait"):
                pltpu.make_async_copy(
                    seq_hbm.at[pl.ds(off, L)], idx_v[b], fsem[b]).wait()
                pltpu.make_async_copy(
                    lbl_hbm.at[pl.ds(off, L)], lbl_v[b], fsem[b]).wait()
            pltpu.async_copy(tok_hbm.at[idx_v[b].at[pl.ds(0, 128)]],
                             rows_v[b].at[pl.ds(0, 128)], gsem[b])
            pltpu.async_copy(tok_hbm.at[idx_v[b].at[pl.ds(128, L - 128)]],
                             rows_v[b].at[pl.ds(128, L - 128)], gsem[b])

        for b in range(NBUF):
            with jax.named_scope("gwait"):
                pltpu.make_async_copy(
                    tok_hbm.at[idx_v[b].at[pl.ds(0, 128)]],
                    rows_v[b].at[pl.ds(0, 128)], gsem[b]).wait()
                pltpu.make_async_copy(
                    tok_hbm.at[idx_v[b].at[pl.ds(128, L - 128)]],
                    rows_v[b].at[pl.ds(128, L - 128)], gsem[b]).wait()

            @plsc.parallel_loop(0, L, unroll=2)
            def restage(r, b=b):
                with jax.named_scope("restage"):
                    rb = r * STRIDE
                    for d in range(DIM // LANES):
                        tok65_v[pl.ds(rb + d * LANES, LANES)] = (
                            rows_v[b][r, pl.ds(d * LANES, LANES)])

            if b == 0:
                @pl.when(k0 > 0)
                def _drain():
                    pltpu.make_async_copy(
                        pan_v, out_hbm.at[rows[0]], ssem).wait()
            else:
                pltpu.make_async_copy(
                    pan_v, out_hbm.at[rows[b]], ssem).wait()

            @pl.loop(0, len(GROUP_BASES))
            def per_group(g, b=b):
                with jax.named_scope("addloop"):
                    gb = lax.min(g * LANES, jnp.int32(L - LANES))
                    sl = pl.ds(gb, LANES)
                    lvec = gb + lax.iota(jnp.int32, LANES)
                    tbase = lvec * STRIDE
                    cbase = lbl_v[b][sl] * (DIM * L) + lvec
                    for dd in range(DIM):
                        t16 = plsc.load_gather(tok65_v, [tbase + dd])
                        c16 = plsc.load_gather(ct_v, [cbase + dd * L])
                        pan_v[dd, sl] = t16 + c16

            pltpu.async_copy(pan_v, out_hbm.at[rows[b]], ssem)

    pltpu.make_async_copy(
        pan_v, out_hbm.at[rbase + ROWS_PW - 1], ssem).wait()


@jax.jit
def _sc_embed(seq_flat, lbl_flat, tok128, ct_flat):
    mesh = plsc.VectorSubcoreMesh(core_axis_name="c", subcore_axis_name="s")
    return pl.kernel(
        _sc_body,
        out_type=jax.ShapeDtypeStruct((B, DIM, L), jnp.float32),
        mesh=mesh,
        compiler_params=pltpu.CompilerParams(
            use_tc_tiling_on_sc=True, needs_layout_passes=False,
            disable_bounds_checks=True),
        scratch_types=(
            [pltpu.VMEM((L,), jnp.int32)] * NBUF
            + [pltpu.VMEM((L,), jnp.int32)] * NBUF
            + [pltpu.VMEM((L, 128), jnp.float32)] * NBUF
            + [pltpu.VMEM((DIM, L), jnp.float32)]
            + [pltpu.VMEM((N_SEG * DIM * L,), jnp.float32)]
            + [pltpu.VMEM((L * STRIDE,), jnp.float32)]
            + [pltpu.SemaphoreType.DMA] * (2 * NBUF + 1)
        ),
    )(seq_flat, lbl_flat, tok128, ct_flat)


def kernel(seq, seg_label, token_table, seg_table):
    pe_t = jnp.asarray(_PE_T)
    ct_flat = _combine(pe_t, seg_table)
    tok128 = _dup(token_table)
    out_dl = _sc_embed(
        seq.reshape(B * L).astype(jnp.int32),
        seg_label.reshape(B * L).astype(jnp.int32),
        tok128,
        ct_flat,
    )
    return out_dl.transpose(0, 2, 1)
